# baseline (speedup 1.0000x reference)
import jax
import jax.numpy as jnp
from jax import lax
from jax.experimental import pallas as pl
from jax.experimental.pallas import tpu as pltpu
from jax.experimental.pallas import tpu_sc as plsc

B = 4096
L = 200
D = 32
NC = 2
NS = 16
NW = NC * NS
BAND = B // NW
NBUF = 8
TBUF = 2


def _body(lines_hbm, table_hbm, out_hbm, lines_v, rows_v, tile_v, gsem, wsem):
    cid = lax.axis_index("c")
    sid = lax.axis_index("s")
    wid = sid * NC + cid
    pltpu.sync_copy(lines_hbm.at[wid], lines_v)

    iota = lax.iota(jnp.int32, 16)
    dt_vecs = [(c + iota) // 8 for c in (0, 16)]
    ds_vecs = [(c + iota) % 8 for c in (0, 16)]

    def start_gather(l, bg):
        pltpu.async_copy(table_hbm.at[lines_v.at[l]], rows_v.at[bg],
                         gsem.at[bg])

    def wait_gather(bg):
        pltpu.make_async_copy(table_hbm.at[lines_v.at[0]], rows_v.at[bg],
                              gsem.at[bg]).wait()

    def start_wb(l, bt):
        pltpu.async_copy(tile_v.at[bt, :, :, pl.ds(0, 128)],
                         out_hbm.at[l, :, wid], wsem.at[bt])

    def wait_wb(bt):
        pltpu.make_async_copy(tile_v.at[bt, :, :, pl.ds(0, 128)],
                              out_hbm.at[0, :, wid], wsem.at[bt]).wait()

    def transpose(bg, bt):
        @plsc.parallel_loop(0, BAND, step=4, unroll=4)
        def tloop(b0):
            vals = [rows_v[bg, b0 + i, pl.ds(c, 16)]
                    for i in range(4) for c in (0, 16)]
            for i in range(4):
                bvec = jnp.full((16,), b0 + i, jnp.int32)
                for h in range(2):
                    plsc.store_scatter(
                        tile_v.at[bt],
                        [dt_vecs[h], ds_vecs[h], bvec],
                        vals[i * 2 + h])

    def iteration(l, bg, bt, do_wait_wb, do_refill):
        wait_gather(bg)
        if do_wait_wb:
            wait_wb(bt)
        transpose(bg, bt)
        start_wb(l, bt)
        if do_refill:
            start_gather(l + NBUF, bg)

    for l in range(NBUF):
        start_gather(l, l)
    for l in range(NBUF):
        iteration(l, l % NBUF, l % TBUF, l >= TBUF, True)

    def round_body(g, _):
        l0 = NBUF + (g - 1) * NBUF
        for k in range(NBUF):
            iteration(l0 + k, k, k % TBUF, True, True)
        return 0

    lax.fori_loop(1, (L - NBUF) // NBUF, round_body, 0)

    for l in range(L - NBUF, L):
        iteration(l, l % NBUF, l % TBUF, True, False)
    for bt in range(TBUF):
        wait_wb(bt)


@jax.jit
def _gather(lines, table):
    run = pl.kernel(
        _body,
        mesh=plsc.VectorSubcoreMesh(core_axis_name="c", subcore_axis_name="s"),
        out_type=jax.ShapeDtypeStruct((L, 4, NW, 8, 128), jnp.float32),
        scratch_types=[
            pltpu.VMEM((L, BAND), jnp.int32),
            pltpu.VMEM((NBUF, BAND, D), jnp.float32),
            pltpu.VMEM((TBUF, 4, 8, 129), jnp.float32),
            pltpu.SemaphoreType.DMA((NBUF,)),
            pltpu.SemaphoreType.DMA((TBUF,)),
        ],
        compiler_params=pltpu.CompilerParams(use_tc_tiling_on_sc=False,
                                             needs_layout_passes=False,
                                             disable_bounds_checks=True),
    )
    return run(lines, table)


def kernel(relation_indices, relation_embedding_weight):
    lines = relation_indices[:, :, 0].reshape(NW, BAND, L).transpose(0, 2, 1)
    out5 = _gather(lines, relation_embedding_weight)
    return out5.transpose(2, 4, 0, 1, 3).reshape(B, L, D)


# speedup vs baseline: 1.0011x; 1.0011x over previous
import jax
import jax.numpy as jnp
from jax import lax
from jax.experimental import pallas as pl
from jax.experimental.pallas import tpu as pltpu
from jax.experimental.pallas import tpu_sc as plsc

B = 4096
L = 200
D = 32
NC = 2
NS = 16
NW = NC * NS
BAND = B // NW
NBUF = 4
TBUF = 2


def _body(lines_hbm, table_hbm, out_hbm, lines_v, rows_v, tile_v, gsem, wsem):
    cid = lax.axis_index("c")
    sid = lax.axis_index("s")
    wid = sid * NC + cid
    pltpu.sync_copy(lines_hbm.at[wid], lines_v)

    iota = lax.iota(jnp.int32, 16)
    dt_vecs = [(c + iota) // 8 for c in (0, 16)]
    ds_vecs = [(c + iota) % 8 for c in (0, 16)]

    def start_gather(l, bg):
        pltpu.async_copy(table_hbm.at[lines_v.at[l]], rows_v.at[bg],
                         gsem.at[bg])

    def wait_gather(bg):
        pltpu.make_async_copy(table_hbm.at[lines_v.at[0]], rows_v.at[bg],
                              gsem.at[bg]).wait()

    def start_wb(l, bt):
        pltpu.async_copy(tile_v.at[bt, :, :, pl.ds(0, 128)],
                         out_hbm.at[l, :, wid], wsem.at[bt])

    def wait_wb(bt):
        pltpu.make_async_copy(tile_v.at[bt, :, :, pl.ds(0, 128)],
                              out_hbm.at[0, :, wid], wsem.at[bt]).wait()

    def transpose(bg, bt):
        @plsc.parallel_loop(0, BAND, step=4, unroll=4)
        def tloop(b0):
            vals = [rows_v[bg, b0 + i, pl.ds(c, 16)]
                    for i in range(4) for c in (0, 16)]
            for i in range(4):
                bvec = jnp.full((16,), b0 + i, jnp.int32)
                for h in range(2):
                    plsc.store_scatter(
                        tile_v.at[bt],
                        [dt_vecs[h], ds_vecs[h], bvec],
                        vals[i * 2 + h])

    def iteration(l, bg, bt, do_wait_wb, do_refill):
        wait_gather(bg)
        if do_wait_wb:
            wait_wb(bt)
        transpose(bg, bt)
        start_wb(l, bt)
        if do_refill:
            start_gather(l + NBUF, bg)

    for l in range(NBUF):
        start_gather(l, l)
    for l in range(NBUF):
        iteration(l, l % NBUF, l % TBUF, l >= TBUF, True)

    def round_body(g, _):
        l0 = NBUF + (g - 1) * NBUF
        for k in range(NBUF):
            iteration(l0 + k, k, k % TBUF, True, True)
        return 0

    lax.fori_loop(1, (L - NBUF) // NBUF, round_body, 0)

    for l in range(L - NBUF, L):
        iteration(l, l % NBUF, l % TBUF, True, False)
    for bt in range(TBUF):
        wait_wb(bt)


@jax.jit
def _gather(lines, table):
    run = pl.kernel(
        _body,
        mesh=plsc.VectorSubcoreMesh(core_axis_name="c", subcore_axis_name="s"),
        out_type=jax.ShapeDtypeStruct((L, 4, NW, 8, 128), jnp.float32),
        scratch_types=[
            pltpu.VMEM((L, BAND), jnp.int32),
            pltpu.VMEM((NBUF, BAND, D), jnp.float32),
            pltpu.VMEM((TBUF, 4, 8, 129), jnp.float32),
            pltpu.SemaphoreType.DMA((NBUF,)),
            pltpu.SemaphoreType.DMA((TBUF,)),
        ],
        compiler_params=pltpu.CompilerParams(use_tc_tiling_on_sc=False,
                                             needs_layout_passes=False,
                                             disable_bounds_checks=True),
    )
    return run(lines, table)


def kernel(relation_indices, relation_embedding_weight):
    lines = relation_indices[:, :, 0].reshape(NW, BAND, L).transpose(0, 2, 1)
    out5 = _gather(lines, relation_embedding_weight)
    return out5.transpose(2, 4, 0, 1, 3).reshape(B, L, D)
